# baseline (speedup 1.0000x reference)
import functools

import jax
import jax.numpy as jnp
from jax import lax
from jax.experimental import pallas as pl
from jax.experimental.pallas import tpu as pltpu
from jax.experimental.pallas import tpu_sc as plsc

_NUM = 13
_CAT = 26
_H = 32
_VOCAB = 100000


_VB = 4096
_QL = _VB // 4


def _tc_detile(tabT):
    V = 100000
    NJ = (V + _VB - 1) // _VB

    def body(in_ref, out_ref):
        z = in_ref[0].T
        out_ref[0] = jnp.concatenate(
            [z[q * _QL:(q + 1) * _QL] for q in range(4)], axis=1)

    return pl.pallas_call(
        body,
        grid=(_CAT, NJ),
        in_specs=[pl.BlockSpec((1, _H, _VB), lambda c, j: (c, 0, j))],
        out_specs=pl.BlockSpec((1, _QL, 128), lambda c, j: (c, j, 0)),
        out_shape=jax.ShapeDtypeStruct((_CAT, NJ * _QL, 128), jnp.float32),
        compiler_params=pltpu.CompilerParams(
            dimension_semantics=("parallel", "parallel")),
    )(tabT)


def _sc_gather(tab_lines, pidx2d):
    NW = 32
    NPOS = 16384 * 28
    per_w = NPOS // NW
    nchunk = per_w // 1024
    NLINE = NPOS // 4
    mesh = plsc.VectorSubcoreMesh(core_axis_name="c", subcore_axis_name="s")

    @functools.partial(
        pl.kernel,
        mesh=mesh,
        out_type=jax.ShapeDtypeStruct((NLINE, 128), jnp.float32),
        scratch_types=[
            pltpu.VMEM((8, 128), jnp.int32),
            pltpu.VMEM((8, 128), jnp.int32),
            pltpu.VMEM((1024,), jnp.int32),
            pltpu.VMEM((512, 128), jnp.float32),
            pltpu.VMEM((128, 128), jnp.float32),
            pltpu.SemaphoreType.DMA,
        ],
        compiler_params=pltpu.CompilerParams(use_tc_tiling_on_sc=False),
    )
    def gather_kernel(tab_hbm, pidx_hbm, out_hbm, idx_v, lidx_v, s32_v,
                      rows_v, obuf_v, sem):
        cid = lax.axis_index("c")
        sid = lax.axis_index("s")
        wid = sid * 2 + cid
        iota = lax.iota(jnp.int32, 16)
        drow_off = iota // 4
        dlane_base = (iota % 4) * 32

        def chunk(k, carry):
            row0 = pl.multiple_of(wid * 112 + k * 8, 8)
            pltpu.sync_copy(pidx_hbm.at[pl.ds(row0, 8)], idx_v)
            for j in range(8):
                for l in range(8):
                    x = idx_v[j, pl.ds(l * 16, 16)]
                    lidx_v[j, pl.ds(l * 16, 16)] = x >> 2
                    s32_v[pl.ds(j * 128 + l * 16, 16)] = (x & 3) * 32
            for half in range(2):
                cps = [
                    pltpu.async_copy(
                        tab_hbm.at[lidx_v.at[half * 4 + j4]],
                        rows_v.at[pl.ds(j4 * 128, 128)],
                        sem,
                    )
                    for j4 in range(4)
                ]
                for c in cps:
                    c.wait()
                def gstep(g, carry2):
                    svec = s32_v[pl.ds(half * 512 + g * 16, 16)]
                    for i in range(16):
                        s = svec[i]
                        p = g * 16 + i
                        v0 = rows_v[p, pl.ds(s, 16)]
                        v1 = rows_v[p, pl.ds(s + 16, 16)]
                        obuf_v[g * 4 + i // 4,
                               pl.ds((i % 4) * 32, 16)] = v0
                        obuf_v[g * 4 + i // 4,
                               pl.ds((i % 4) * 32 + 16, 16)] = v1
                    return carry2

                lax.fori_loop(0, 32, gstep, 0)
                lo = pl.multiple_of(wid * 3584 + k * 256 + half * 128, 8)
                pltpu.sync_copy(obuf_v, out_hbm.at[pl.ds(lo, 128)])
            return carry

        lax.fori_loop(0, nchunk, chunk, 0)

    return gather_kernel(tab_lines, pidx2d)


def _tc_bn(x, W_exp, b_flat, cat2d, g_n, bt_n, g_c, bt_c):
    B = x.shape[0]
    DN = W_exp.shape[1]
    DC = 832
    DCP = cat2d.shape[1]
    NB = 16
    Bb = B // NB
    inv_b = 1.0 / B

    def body(x_ref, w_ref, b_ref, cat_ref, gn_ref, bn_ref, gc_ref, bc_ref,
             out_ref, stn_ref, stc_ref):
        p = pl.program_id(0)
        i = pl.program_id(1)
        xb = x_ref[...]
        en = jnp.maximum(
            jnp.dot(xb, w_ref[...], preferred_element_type=jnp.float32)
            + b_ref[...], 0.0)
        cb = cat_ref[...][:, :DC]

        @pl.when(jnp.logical_and(p == 0, i == 0))
        def _init():
            stn_ref[...] = jnp.zeros_like(stn_ref)
            stc_ref[...] = jnp.zeros_like(stc_ref)

        @pl.when(p == 0)
        def _stats():
            stn_ref[0:1, :] += jnp.sum(en, axis=0, keepdims=True)
            stn_ref[1:2, :] += jnp.sum(en * en, axis=0, keepdims=True)
            stc_ref[0:1, :] += jnp.sum(cb, axis=0, keepdims=True)
            stc_ref[1:2, :] += jnp.sum(cb * cb, axis=0, keepdims=True)

        @pl.when(jnp.logical_and(p == 0, i == NB - 1))
        def _finalize():
            mean_n = stn_ref[0:1, :] * inv_b
            var_n = stn_ref[1:2, :] * inv_b - mean_n * mean_n
            sc_n = gn_ref[...] * lax.rsqrt(var_n + 1e-5)
            stn_ref[2:3, :] = sc_n
            stn_ref[3:4, :] = bn_ref[...] - mean_n * sc_n
            mean_c = stc_ref[0:1, :] * inv_b
            var_c = stc_ref[1:2, :] * inv_b - mean_c * mean_c
            sc_c = gc_ref[...] * lax.rsqrt(var_c + 1e-5)
            stc_ref[2:3, :] = sc_c
            stc_ref[3:4, :] = bc_ref[...] - mean_c * sc_c

        @pl.when(p == 1)
        def _write():
            out_ref[:, :DN] = en * stn_ref[2:3, :] + stn_ref[3:4, :]
            out_ref[:, DN:] = cb * stc_ref[2:3, :] + stc_ref[3:4, :]

    return pl.pallas_call(
        body,
        grid=(2, NB),
        in_specs=[
            pl.BlockSpec((Bb, _NUM), lambda p, i: (i, 0)),
            pl.BlockSpec((_NUM, DN), lambda p, i: (0, 0)),
            pl.BlockSpec((1, DN), lambda p, i: (0, 0)),
            pl.BlockSpec((Bb, DCP), lambda p, i: (i, 0)),
            pl.BlockSpec((1, DN), lambda p, i: (0, 0)),
            pl.BlockSpec((1, DN), lambda p, i: (0, 0)),
            pl.BlockSpec((1, DC), lambda p, i: (0, 0)),
            pl.BlockSpec((1, DC), lambda p, i: (0, 0)),
        ],
        out_specs=pl.BlockSpec(
            (Bb, DN + DC), lambda p, i: (jnp.where(p == 0, 0, i), 0)),
        out_shape=jax.ShapeDtypeStruct((B, DN + DC), jnp.float32),
        scratch_shapes=[
            pltpu.VMEM((4, DN), jnp.float32),
            pltpu.VMEM((4, DC), jnp.float32),
        ],
        compiler_params=pltpu.CompilerParams(
            dimension_semantics=("arbitrary", "arbitrary")),
    )(x, W_exp, b_flat, cat2d, g_n, bt_n, g_c, bt_c)


def kernel(input_data, num_W, num_b, cat_tables, bn_gamma, bn_beta):
    B = input_data.shape[0]
    x = input_data[:, :_NUM]
    idx = input_data[:, _NUM:].astype(jnp.int32)
    line = ((jnp.arange(_CAT, dtype=jnp.int32) * 25600)[None, :]
            + (idx // _VB) * _QL + idx % _QL)
    pidx = line * 4 + (idx % _VB) // _QL
    pidx28 = jnp.concatenate(
        [pidx, jnp.zeros((B, 2), jnp.int32)], axis=1)
    pidx2d = pidx28.reshape(B * 28 // 128, 128)

    tab_lines = _tc_detile(cat_tables.transpose(0, 2, 1))
    tab_flat = tab_lines.reshape(_CAT * 25600, 128)

    emb = _sc_gather(tab_flat, pidx2d)
    cat2d = emb.reshape(B, 28 * _H)

    DN = _NUM * _H
    W_exp = (num_W[:, None, :]
             * jnp.eye(_NUM, dtype=jnp.float32)[:, :, None]).reshape(_NUM, DN)
    b_flat = num_b.reshape(1, DN)
    g_n = bn_gamma[:DN].reshape(1, DN)
    bt_n = bn_beta[:DN].reshape(1, DN)
    g_c = bn_gamma[DN:].reshape(1, _CAT * _H)
    bt_c = bn_beta[DN:].reshape(1, _CAT * _H)

    out2d = _tc_bn(x, W_exp, b_flat, cat2d, g_n, bt_n, g_c, bt_c)
    return out2d.reshape(B, _NUM + _CAT, _H)
